# baseline (speedup 1.0000x reference)
import functools
import jax
import jax.numpy as jnp
from jax import lax
from jax.experimental import pallas as pl
from jax.experimental.pallas import tpu as pltpu
from jax.experimental.pallas import tpu_sc as plsc

D_MODEL = 1024
MONTH_ROWS = 13
HOUR_ROWS = 25
COMB_ROWS = MONTH_ROWS * HOUR_ROWS
NC = 2
NS = 16
NW = NC * NS
L = 16

N_TOTAL = 4 * 4096
ROWS_PER_W = N_TOTAL // NW
CHUNK = 32
N_CHUNKS = ROWS_PER_W // CHUNK
GROUPS = D_MODEL // L
NBUF = 3


def _sc_kernel(m_idx_hbm, h_idx_hbm, month_hbm, hour_hbm, out_hbm,
               m_idx_v, h_idx_v, cidx_v, mrow_v, hrow_v, rowbuf_v,
               buf0, buf1, buf2, comb_hbm,
               gsem0, gsem1, gsem2, osem0, osem1, osem2):
    cid = lax.axis_index("c")
    sid = lax.axis_index("s")
    wid = sid * NC + cid
    base = wid * ROWS_PER_W

    @pl.when(sid < MONTH_ROWS)
    def _build():
        pltpu.sync_copy(month_hbm.at[sid], mrow_v)

        def jbody(j, carry):
            pltpu.sync_copy(hour_hbm.at[j], hrow_v)
            for g in range(GROUPS):
                sl = pl.ds(g * L, L)
                rowbuf_v[sl] = hrow_v[sl] + mrow_v[sl]
            pltpu.sync_copy(
                rowbuf_v, comb_hbm.at[cid * COMB_ROWS + sid * HOUR_ROWS + j])
            return carry

        lax.fori_loop(0, HOUR_ROWS, jbody, 0)

    pltpu.sync_copy(m_idx_hbm.at[wid], m_idx_v)
    pltpu.sync_copy(h_idx_hbm.at[wid], h_idx_v)
    comb_base = cid * COMB_ROWS
    for c in range(N_CHUNKS):
        for q in range(CHUNK // L):
            sl = pl.ds(q * L, L)
            cidx_v[c, sl] = (m_idx_v[c, sl] * HOUR_ROWS + h_idx_v[c, sl]
                             + comb_base)

    plsc.subcore_barrier()

    bufs = (buf0, buf1, buf2)
    gsems = (gsem0, gsem1, gsem2)
    osems = (osem0, osem1, osem2)
    gat_d = [None] * NBUF
    out_d = [None] * NBUF

    for c in range(NBUF):
        gat_d[c] = pltpu.async_copy(
            comb_hbm.at[cidx_v.at[c]], bufs[c], gsems[c])
    for c in range(N_CHUNKS):
        b = c % NBUF
        gat_d[b].wait()
        out_d[b] = pltpu.async_copy(
            bufs[b], out_hbm.at[pl.ds(base + c * CHUNK, CHUNK)], osems[b])
        if c + NBUF < N_CHUNKS:
            out_d[b].wait()
            gat_d[b] = pltpu.async_copy(
                comb_hbm.at[cidx_v.at[c + NBUF]], bufs[b], gsems[b])
    for c in range(N_CHUNKS - NBUF, N_CHUNKS):
        out_d[c % NBUF].wait()


@jax.jit
def _run(m_idx, h_idx, month_table, hour_table):
    mesh = plsc.VectorSubcoreMesh(core_axis_name="c", subcore_axis_name="s")
    k = functools.partial(
        pl.kernel,
        out_type=jax.ShapeDtypeStruct((N_TOTAL, D_MODEL), jnp.float32),
        mesh=mesh,
        scratch_types=[
            pltpu.VMEM((N_CHUNKS, CHUNK), jnp.int32),
            pltpu.VMEM((N_CHUNKS, CHUNK), jnp.int32),
            pltpu.VMEM((N_CHUNKS, CHUNK), jnp.int32),
            pltpu.VMEM((D_MODEL,), jnp.float32),
            pltpu.VMEM((D_MODEL,), jnp.float32),
            pltpu.VMEM((D_MODEL,), jnp.float32),
            pltpu.VMEM((CHUNK, D_MODEL), jnp.float32),
            pltpu.VMEM((CHUNK, D_MODEL), jnp.float32),
            pltpu.VMEM((CHUNK, D_MODEL), jnp.float32),
            pltpu.HBM((NC * COMB_ROWS, D_MODEL), jnp.float32),
            pltpu.SemaphoreType.DMA,
            pltpu.SemaphoreType.DMA,
            pltpu.SemaphoreType.DMA,
            pltpu.SemaphoreType.DMA,
            pltpu.SemaphoreType.DMA,
            pltpu.SemaphoreType.DMA,
        ],
    )(_sc_kernel)
    return k(m_idx, h_idx, month_table, hour_table)


def kernel(x, hour_table, month_table, minute_table):
    xi = x.astype(jnp.int32).reshape(N_TOTAL, 2)
    m_idx = xi[:, 0].reshape(NW, N_CHUNKS, CHUNK)
    h_idx = xi[:, 1].reshape(NW, N_CHUNKS, CHUNK)
    out = _run(m_idx, h_idx, month_table, hour_table)
    return out.reshape(4, 4096, D_MODEL)


# speedup vs baseline: 1.0973x; 1.0973x over previous
import functools
import jax
import jax.numpy as jnp
from jax import lax
from jax.experimental import pallas as pl
from jax.experimental.pallas import tpu as pltpu
from jax.experimental.pallas import tpu_sc as plsc

D_MODEL = 1024
MONTH_ROWS = 13
HOUR_ROWS = 25
COMB_ROWS = MONTH_ROWS * HOUR_ROWS
NC = 2
NS = 16
NW = NC * NS
L = 16

N_TOTAL = 4 * 4096
ROWS_PER_W = N_TOTAL // NW
CHUNK = 32
N_CHUNKS = ROWS_PER_W // CHUNK
GROUPS = D_MODEL // L
NBUF = 3


def _sc_kernel(m_idx_hbm, h_idx_hbm, month_hbm, hour_hbm, out_hbm,
               m_idx_v, h_idx_v, cidx_v, mrow_v, rowbuf_v,
               buf0, buf1, buf2, comb_hbm,
               gsem0, gsem1, gsem2, osem0, osem1, osem2):
    cid = lax.axis_index("c")
    sid = lax.axis_index("s")
    wid = sid * NC + cid
    base = wid * ROWS_PER_W

    @pl.when(sid < MONTH_ROWS)
    def _build():
        pltpu.sync_copy(month_hbm.at[sid], mrow_v)
        pltpu.sync_copy(hour_hbm, buf0)

        def jbody(j, carry):
            for g in range(GROUPS):
                sl = pl.ds(g * L, L)
                rowbuf_v[sl] = buf0[j, sl] + mrow_v[sl]
            pltpu.sync_copy(
                rowbuf_v, comb_hbm.at[cid * COMB_ROWS + sid * HOUR_ROWS + j])
            return carry

        lax.fori_loop(0, HOUR_ROWS, jbody, 0)

    pltpu.sync_copy(m_idx_hbm.at[wid], m_idx_v)
    pltpu.sync_copy(h_idx_hbm.at[wid], h_idx_v)
    comb_base = cid * COMB_ROWS
    for c in range(N_CHUNKS):
        for q in range(CHUNK // L):
            sl = pl.ds(q * L, L)
            cidx_v[c, sl] = (m_idx_v[c, sl] * HOUR_ROWS + h_idx_v[c, sl]
                             + comb_base)

    plsc.subcore_barrier()

    bufs = (buf0, buf1, buf2)
    gsems = (gsem0, gsem1, gsem2)
    osems = (osem0, osem1, osem2)
    gat_d = [None] * NBUF
    out_d = [None] * NBUF

    for c in range(NBUF):
        gat_d[c] = pltpu.async_copy(
            comb_hbm.at[cidx_v.at[c]], bufs[c], gsems[c])
    for c in range(N_CHUNKS):
        b = c % NBUF
        gat_d[b].wait()
        out_d[b] = pltpu.async_copy(
            bufs[b], out_hbm.at[pl.ds(base + c * CHUNK, CHUNK)], osems[b])
        if c + NBUF < N_CHUNKS:
            out_d[b].wait()
            gat_d[b] = pltpu.async_copy(
                comb_hbm.at[cidx_v.at[c + NBUF]], bufs[b], gsems[b])
    for c in range(N_CHUNKS - NBUF, N_CHUNKS):
        out_d[c % NBUF].wait()


@jax.jit
def _run(m_idx, h_idx, month_table, hour_table):
    mesh = plsc.VectorSubcoreMesh(core_axis_name="c", subcore_axis_name="s")
    k = functools.partial(
        pl.kernel,
        out_type=jax.ShapeDtypeStruct((N_TOTAL, D_MODEL), jnp.float32),
        mesh=mesh,
        scratch_types=[
            pltpu.VMEM((N_CHUNKS, CHUNK), jnp.int32),
            pltpu.VMEM((N_CHUNKS, CHUNK), jnp.int32),
            pltpu.VMEM((N_CHUNKS, CHUNK), jnp.int32),
            pltpu.VMEM((D_MODEL,), jnp.float32),
            pltpu.VMEM((D_MODEL,), jnp.float32),
            pltpu.VMEM((CHUNK, D_MODEL), jnp.float32),
            pltpu.VMEM((CHUNK, D_MODEL), jnp.float32),
            pltpu.VMEM((CHUNK, D_MODEL), jnp.float32),
            pltpu.HBM((NC * COMB_ROWS, D_MODEL), jnp.float32),
            pltpu.SemaphoreType.DMA,
            pltpu.SemaphoreType.DMA,
            pltpu.SemaphoreType.DMA,
            pltpu.SemaphoreType.DMA,
            pltpu.SemaphoreType.DMA,
            pltpu.SemaphoreType.DMA,
        ],
    )(_sc_kernel)
    return k(m_idx, h_idx, month_table, hour_table)


def kernel(x, hour_table, month_table, minute_table):
    xi = x.astype(jnp.int32).reshape(N_TOTAL, 2)
    m_idx = xi[:, 0].reshape(NW, N_CHUNKS, CHUNK)
    h_idx = xi[:, 1].reshape(NW, N_CHUNKS, CHUNK)
    hour32 = jnp.pad(hour_table, ((0, CHUNK - HOUR_ROWS), (0, 0)))
    out = _run(m_idx, h_idx, month_table, hour32)
    return out.reshape(4, 4096, D_MODEL)


# speedup vs baseline: 1.2221x; 1.1137x over previous
import functools
import jax
import jax.numpy as jnp
from jax import lax
from jax.experimental import pallas as pl
from jax.experimental.pallas import tpu as pltpu
from jax.experimental.pallas import tpu_sc as plsc

D_MODEL = 1024
MONTH_ROWS = 13
HOUR_ROWS = 25
COMB_ROWS = MONTH_ROWS * HOUR_ROWS
NC = 2
NS = 16
NW = NC * NS
L = 16

N_TOTAL = 4 * 4096
ROWS_PER_W = N_TOTAL // NW
CHUNK = 32
N_CHUNKS = ROWS_PER_W // CHUNK
NBUF = 3


def _build_kernel(month_ref, hour_ref, comb_ref):
    m = month_ref[...].reshape(MONTH_ROWS, 1, D_MODEL)
    h = hour_ref[...].reshape(1, HOUR_ROWS, D_MODEL)
    comb_ref[...] = (m + h).reshape(COMB_ROWS, D_MODEL)


def _sc_kernel(cidx_hbm, comb_hbm, out_hbm, cidx_v,
               buf0, buf1, buf2, gsem0, gsem1, gsem2, osem0, osem1, osem2):
    cid = lax.axis_index("c")
    sid = lax.axis_index("s")
    wid = sid * NC + cid
    base = wid * ROWS_PER_W

    pltpu.sync_copy(cidx_hbm.at[wid], cidx_v)

    bufs = (buf0, buf1, buf2)
    gsems = (gsem0, gsem1, gsem2)
    osems = (osem0, osem1, osem2)
    gat_d = [None] * NBUF
    out_d = [None] * NBUF

    for c in range(NBUF):
        gat_d[c] = pltpu.async_copy(
            comb_hbm.at[cidx_v.at[c]], bufs[c], gsems[c])
    for c in range(N_CHUNKS):
        b = c % NBUF
        gat_d[b].wait()
        out_d[b] = pltpu.async_copy(
            bufs[b], out_hbm.at[pl.ds(base + c * CHUNK, CHUNK)], osems[b])
        if c + NBUF < N_CHUNKS:
            out_d[b].wait()
            gat_d[b] = pltpu.async_copy(
                comb_hbm.at[cidx_v.at[c + NBUF]], bufs[b], gsems[b])
    for c in range(N_CHUNKS - NBUF, N_CHUNKS):
        out_d[c % NBUF].wait()


@jax.jit
def _run(cidx, month_table, hour_table):
    comb = pl.pallas_call(
        _build_kernel,
        out_shape=jax.ShapeDtypeStruct((COMB_ROWS, D_MODEL), jnp.float32),
    )(month_table, hour_table)

    mesh = plsc.VectorSubcoreMesh(core_axis_name="c", subcore_axis_name="s")
    k = functools.partial(
        pl.kernel,
        out_type=jax.ShapeDtypeStruct((N_TOTAL, D_MODEL), jnp.float32),
        mesh=mesh,
        scratch_types=[
            pltpu.VMEM((N_CHUNKS, CHUNK), jnp.int32),
            pltpu.VMEM((CHUNK, D_MODEL), jnp.float32),
            pltpu.VMEM((CHUNK, D_MODEL), jnp.float32),
            pltpu.VMEM((CHUNK, D_MODEL), jnp.float32),
            pltpu.SemaphoreType.DMA,
            pltpu.SemaphoreType.DMA,
            pltpu.SemaphoreType.DMA,
            pltpu.SemaphoreType.DMA,
            pltpu.SemaphoreType.DMA,
            pltpu.SemaphoreType.DMA,
        ],
    )(_sc_kernel)
    return k(cidx, comb)


def kernel(x, hour_table, month_table, minute_table):
    xi = x.astype(jnp.int32).reshape(N_TOTAL, 2)
    cidx = (xi[:, 0] * HOUR_ROWS + xi[:, 1]).reshape(NW, N_CHUNKS, CHUNK)
    out = _run(cidx, month_table, hour_table)
    return out.reshape(4, 4096, D_MODEL)
